# baseline (speedup 1.0000x reference)
import functools

import jax
import jax.numpy as jnp
from jax import lax
from jax.experimental import pallas as pl
from jax.experimental.pallas import tpu as pltpu
from jax.experimental.pallas import tpu_sc as plsc

N_NODES = 10000
NP = 10240
N_EDGES = 320000
CHUNK = 128
N_CHUNKS = 2560
EP = N_CHUNKS * CHUNK
CPT = 80
DH = 64
DW = 128
ROWS_PER_TILE = NP // 16

_mesh = plsc.VectorSubcoreMesh(core_axis_name="c", subcore_axis_name="s")


def _deg_body(dstc_hbm, out_hbm, dst_v, ones_v, deg_sh):
    c = lax.axis_index("c")
    s = lax.axis_index("s")
    wid = c * 16 + s

    def fill_zero(i, carry):
        for k in range(DW // 16):
            ones_v[i, pl.ds(k * 16, 16)] = jnp.zeros((16,), jnp.float32)
        return carry

    lax.fori_loop(0, CHUNK, fill_zero, 0)
    for q in range(ROWS_PER_TILE // CHUNK):
        pltpu.sync_copy(
            ones_v, deg_sh.at[pl.ds(s * ROWS_PER_TILE + q * CHUNK, CHUNK)])

    def fill_ones(i, carry):
        for k in range(DW // 16):
            ones_v[i, pl.ds(k * 16, 16)] = jnp.ones((16,), jnp.float32)
        return carry

    lax.fori_loop(0, CHUNK, fill_ones, 0)
    pltpu.sync_copy(dstc_hbm.at[pl.ds(wid * CPT, CPT)], dst_v)
    plsc.subcore_barrier()

    def body(j, carry):
        pltpu.sync_copy(ones_v, deg_sh.at[dst_v.at[j]], add=True)
        return carry

    lax.fori_loop(0, CPT, body, 0)
    plsc.subcore_barrier()
    pltpu.sync_copy(deg_sh.at[pl.ds(s * ROWS_PER_TILE, ROWS_PER_TILE)],
                    out_hbm.at[c, pl.ds(s * ROWS_PER_TILE, ROWS_PER_TILE)])


_deg_kernel = functools.partial(
    pl.kernel,
    out_type=jax.ShapeDtypeStruct((2, NP, DW), jnp.float32),
    mesh=_mesh,
    scratch_types=[
        pltpu.VMEM((CPT, CHUNK), jnp.int32),
        pltpu.VMEM((CHUNK, DW), jnp.float32),
        pltpu.VMEM_SHARED((NP, DW), jnp.float32),
    ],
)(_deg_body)


def _agg_body(hs_hbm, srcc_hbm, dstc_hbm, out_hbm,
              src_v, dst_v, rows_v, acc_sh, sem):
    c = lax.axis_index("c")
    s = lax.axis_index("s")
    wid = (1 - c) * 16 + s

    def fill_zero(i, carry):
        for k in range(DW // 16):
            rows_v[i, pl.ds(k * 16, 16)] = jnp.zeros((16,), jnp.float32)
        return carry

    lax.fori_loop(0, CHUNK, fill_zero, 0)

    pltpu.sync_copy(srcc_hbm.at[pl.ds(wid * CPT, CPT)], src_v)
    pltpu.sync_copy(dstc_hbm.at[pl.ds(wid * CPT, CPT)], dst_v)
    for q in range(ROWS_PER_TILE // CHUNK):
        pltpu.sync_copy(
            rows_v, acc_sh.at[pl.ds(s * ROWS_PER_TILE + q * CHUNK, CHUNK)])
    plsc.subcore_barrier()

    def body(j, carry):
        pltpu.async_copy(hs_hbm.at[src_v.at[j]], rows_v, sem).wait()
        pltpu.sync_copy(rows_v, acc_sh.at[dst_v.at[j]], add=True)
        return carry

    lax.fori_loop(0, CPT, body, 0)
    plsc.subcore_barrier()
    pltpu.sync_copy(acc_sh.at[pl.ds(s * ROWS_PER_TILE, ROWS_PER_TILE)],
                    out_hbm.at[c, pl.ds(s * ROWS_PER_TILE, ROWS_PER_TILE)])


_agg_kernel = functools.partial(
    pl.kernel,
    out_type=jax.ShapeDtypeStruct((2, NP, DW), jnp.float32),
    mesh=_mesh,
    scratch_types=[
        pltpu.VMEM((CPT, CHUNK), jnp.int32),
        pltpu.VMEM((CPT, CHUNK), jnp.int32),
        pltpu.VMEM((CHUNK, DW), jnp.float32),
        pltpu.VMEM_SHARED((NP, DW), jnp.float32),
        pltpu.SemaphoreType.DMA,
    ],
)(_agg_body)


_GRID = 8
_BR = NP // _GRID


def _dinv(d0, d1):
    return lax.rsqrt(d0[:, :DH] + d1[:, :DH] + 1.0)


def _padw(h):
    return jnp.concatenate([h, jnp.zeros_like(h)], axis=1)


def _tc1_body(x_ref, w1_ref, d0_ref, d1_ref, hs_ref):
    dinv = _dinv(d0_ref[...], d1_ref[...])
    h = jnp.dot(x_ref[...], w1_ref[...], preferred_element_type=jnp.float32)
    hs_ref[...] = _padw(h * dinv)


def _tc2_body(p0_ref, p1_ref, hs1_ref, d0_ref, d1_ref, w2_ref, b1_ref, hs2_ref):
    dinv = _dinv(d0_ref[...], d1_ref[...])
    tot = (p0_ref[...] + p1_ref[...] + hs1_ref[...])[:, :DH]
    o1 = jnp.maximum(tot * dinv + b1_ref[...], 0.0)
    hs2_ref[...] = _padw(
        jnp.dot(o1, w2_ref[...], preferred_element_type=jnp.float32) * dinv)


def _tc3_body(p0_ref, p1_ref, hs2_ref, d0_ref, d1_ref, w3_ref, b2_ref, b3_ref,
              out_ref):
    dinv = _dinv(d0_ref[...], d1_ref[...])
    tot = (p0_ref[...] + p1_ref[...] + hs2_ref[...])[:, :DH]
    o2 = jnp.maximum(tot * dinv + b2_ref[...], 0.0)
    out_ref[...] = jnp.dot(
        o2, w3_ref[...], preferred_element_type=jnp.float32) + b3_ref[...]


def _row_spec(width):
    return pl.BlockSpec((_BR, width), lambda i: (i, 0))


def _full_spec(shape):
    return pl.BlockSpec(shape, lambda i: (0,) * len(shape))


def kernel(x, edge_index, W1, b1, W2, b2, W3, b3):
    src = jnp.concatenate(
        [edge_index[0], jnp.zeros((EP - N_EDGES,), jnp.int32)]
    ).reshape(N_CHUNKS, CHUNK)
    pad_dst = N_NODES + (
        jnp.arange(EP - N_EDGES, dtype=jnp.int32) % (NP - N_NODES))
    dst = jnp.concatenate(
        [edge_index[1], pad_dst]).reshape(N_CHUNKS, CHUNK)
    xp = jnp.pad(x, ((0, NP - N_NODES), (0, 0)))
    b1r = b1.reshape(1, DH)
    b2r = b2.reshape(1, DH)
    b3r = b3.reshape(1, W3.shape[1])

    degb = _deg_kernel(dst)
    d0, d1 = degb[0], degb[1]

    hs1 = pl.pallas_call(
        _tc1_body,
        grid=(_GRID,),
        in_specs=[_row_spec(x.shape[1]), _full_spec(W1.shape),
                  _row_spec(DW), _row_spec(DW)],
        out_specs=_row_spec(DW),
        out_shape=jax.ShapeDtypeStruct((NP, DW), jnp.float32),
    )(xp, W1, d0, d1)

    p1 = _agg_kernel(hs1, src, dst)

    hs2 = pl.pallas_call(
        _tc2_body,
        grid=(_GRID,),
        in_specs=[_row_spec(DW), _row_spec(DW), _row_spec(DW),
                  _row_spec(DW), _row_spec(DW),
                  _full_spec(W2.shape), _full_spec(b1r.shape)],
        out_specs=_row_spec(DW),
        out_shape=jax.ShapeDtypeStruct((NP, DW), jnp.float32),
    )(p1[0], p1[1], hs1, d0, d1, W2, b1r)

    p2 = _agg_kernel(hs2, src, dst)

    out = pl.pallas_call(
        _tc3_body,
        grid=(_GRID,),
        in_specs=[_row_spec(DW), _row_spec(DW), _row_spec(DW),
                  _row_spec(DW), _row_spec(DW),
                  _full_spec(W3.shape), _full_spec(b2r.shape),
                  _full_spec(b3r.shape)],
        out_specs=_row_spec(W3.shape[1]),
        out_shape=jax.ShapeDtypeStruct((NP, W3.shape[1]), jnp.float32),
    )(p2[0], p2[1], hs2, d0, d1, W3, b2r, b3r)

    return out[:N_NODES]


# speedup vs baseline: 2.4321x; 2.4321x over previous
import functools

import jax
import jax.numpy as jnp
from jax import lax
from jax.experimental import pallas as pl
from jax.experimental.pallas import tpu as pltpu
from jax.experimental.pallas import tpu_sc as plsc

N_NODES = 10000
NP = 10240
N_EDGES = 320000
CHUNK = 128
N_CHUNKS = 2560
EP = N_CHUNKS * CHUNK
CPT = 80
DH = 64
DW = 128
ROWS_PER_TILE = NP // 16

_mesh = plsc.VectorSubcoreMesh(core_axis_name="c", subcore_axis_name="s")


def _deg_body(dstc_hbm, out_hbm, dst_v, ones_v, deg_sh):
    c = lax.axis_index("c")
    s = lax.axis_index("s")
    wid = c * 16 + s

    def fill_zero(i, carry):
        for k in range(DW // 16):
            ones_v[i, pl.ds(k * 16, 16)] = jnp.zeros((16,), jnp.float32)
        return carry

    lax.fori_loop(0, CHUNK, fill_zero, 0)
    for q in range(ROWS_PER_TILE // CHUNK):
        pltpu.sync_copy(
            ones_v, deg_sh.at[pl.ds(s * ROWS_PER_TILE + q * CHUNK, CHUNK)])

    def fill_ones(i, carry):
        for k in range(DW // 16):
            ones_v[i, pl.ds(k * 16, 16)] = jnp.ones((16,), jnp.float32)
        return carry

    lax.fori_loop(0, CHUNK, fill_ones, 0)
    pltpu.sync_copy(dstc_hbm.at[pl.ds(wid * CPT, CPT)], dst_v)
    plsc.subcore_barrier()

    def body(j, carry):
        pltpu.sync_copy(ones_v, deg_sh.at[dst_v.at[j]], add=True)
        return carry

    lax.fori_loop(0, CPT, body, 0)
    plsc.subcore_barrier()
    pltpu.sync_copy(deg_sh.at[pl.ds(s * ROWS_PER_TILE, ROWS_PER_TILE)],
                    out_hbm.at[c, pl.ds(s * ROWS_PER_TILE, ROWS_PER_TILE)])


_deg_kernel = functools.partial(
    pl.kernel,
    out_type=jax.ShapeDtypeStruct((2, NP, DW), jnp.float32),
    mesh=_mesh,
    scratch_types=[
        pltpu.VMEM((CPT, CHUNK), jnp.int32),
        pltpu.VMEM((CHUNK, DW), jnp.float32),
        pltpu.VMEM_SHARED((NP, DW), jnp.float32),
    ],
)(_deg_body)


def _agg_body(hs_hbm, srcc_hbm, dstc_hbm, out_hbm,
              src_v, dst_v, rows_v, acc_sh, sem):
    c = lax.axis_index("c")
    s = lax.axis_index("s")
    wid = c * 16 + s

    def fill_zero(i, carry):
        for k in range(DW // 16):
            rows_v[i, pl.ds(k * 16, 16)] = jnp.zeros((16,), jnp.float32)
        return carry

    lax.fori_loop(0, CHUNK, fill_zero, 0)

    pltpu.sync_copy(srcc_hbm.at[pl.ds(wid * CPT, CPT)], src_v)
    pltpu.sync_copy(dstc_hbm.at[pl.ds(wid * CPT, CPT)], dst_v)
    for q in range(ROWS_PER_TILE // CHUNK):
        pltpu.sync_copy(
            rows_v, acc_sh.at[pl.ds(s * ROWS_PER_TILE + q * CHUNK, CHUNK)])
    plsc.subcore_barrier()

    def body(j, carry):
        pltpu.async_copy(hs_hbm.at[src_v.at[j]], rows_v, sem).wait()
        pltpu.sync_copy(rows_v, acc_sh.at[dst_v.at[j]], add=True)
        return carry

    lax.fori_loop(0, CPT, body, 0)
    plsc.subcore_barrier()
    pltpu.sync_copy(acc_sh.at[pl.ds(s * ROWS_PER_TILE, ROWS_PER_TILE)],
                    out_hbm.at[c, pl.ds(s * ROWS_PER_TILE, ROWS_PER_TILE)])


_agg_kernel = functools.partial(
    pl.kernel,
    out_type=jax.ShapeDtypeStruct((2, NP, DW), jnp.float32),
    mesh=_mesh,
    scratch_types=[
        pltpu.VMEM((CPT, CHUNK), jnp.int32),
        pltpu.VMEM((CPT, CHUNK), jnp.int32),
        pltpu.VMEM((CHUNK, DW), jnp.float32),
        pltpu.VMEM_SHARED((NP, DW), jnp.float32),
        pltpu.SemaphoreType.DMA,
    ],
)(_agg_body)


_GRID = 8
_BR = NP // _GRID


def _dinv(d0, d1):
    return lax.rsqrt(d0[:, :DH] + d1[:, :DH] + 1.0)


def _padw(h):
    return jnp.concatenate([h, jnp.zeros_like(h)], axis=1)


def _tc1_body(x_ref, w1_ref, d0_ref, d1_ref, hs_ref):
    dinv = _dinv(d0_ref[...], d1_ref[...])
    h = jnp.dot(x_ref[...], w1_ref[...], preferred_element_type=jnp.float32)
    hs_ref[...] = _padw(h * dinv)


def _tc2_body(p0_ref, p1_ref, hs1_ref, d0_ref, d1_ref, w2_ref, b1_ref, hs2_ref):
    dinv = _dinv(d0_ref[...], d1_ref[...])
    tot = (p0_ref[...] + p1_ref[...] + hs1_ref[...])[:, :DH]
    o1 = jnp.maximum(tot * dinv + b1_ref[...], 0.0)
    hs2_ref[...] = _padw(
        jnp.dot(o1, w2_ref[...], preferred_element_type=jnp.float32) * dinv)


def _tc3_body(p0_ref, p1_ref, hs2_ref, d0_ref, d1_ref, w3_ref, b2_ref, b3_ref,
              out_ref):
    dinv = _dinv(d0_ref[...], d1_ref[...])
    tot = (p0_ref[...] + p1_ref[...] + hs2_ref[...])[:, :DH]
    o2 = jnp.maximum(tot * dinv + b2_ref[...], 0.0)
    out_ref[...] = jnp.dot(
        o2, w3_ref[...], preferred_element_type=jnp.float32) + b3_ref[...]


def _row_spec(width):
    return pl.BlockSpec((_BR, width), lambda i: (i, 0))


def _full_spec(shape):
    return pl.BlockSpec(shape, lambda i: (0,) * len(shape))


def kernel(x, edge_index, W1, b1, W2, b2, W3, b3):
    pad_idx = N_NODES + (
        jnp.arange(EP - N_EDGES, dtype=jnp.int32) % (NP - N_NODES))
    src = jnp.concatenate(
        [edge_index[0], pad_idx]).reshape(N_CHUNKS, CHUNK)
    dst = jnp.concatenate(
        [edge_index[1], pad_idx]).reshape(N_CHUNKS, CHUNK)
    xp = jnp.pad(x, ((0, NP - N_NODES), (0, 0)))
    b1r = b1.reshape(1, DH)
    b2r = b2.reshape(1, DH)
    b3r = b3.reshape(1, W3.shape[1])

    degb = _deg_kernel(dst)
    d0, d1 = degb[0], degb[1]

    hs1 = pl.pallas_call(
        _tc1_body,
        grid=(_GRID,),
        in_specs=[_row_spec(x.shape[1]), _full_spec(W1.shape),
                  _row_spec(DW), _row_spec(DW)],
        out_specs=_row_spec(DW),
        out_shape=jax.ShapeDtypeStruct((NP, DW), jnp.float32),
    )(xp, W1, d0, d1)

    p1 = _agg_kernel(hs1, src, dst)

    hs2 = pl.pallas_call(
        _tc2_body,
        grid=(_GRID,),
        in_specs=[_row_spec(DW), _row_spec(DW), _row_spec(DW),
                  _row_spec(DW), _row_spec(DW),
                  _full_spec(W2.shape), _full_spec(b1r.shape)],
        out_specs=_row_spec(DW),
        out_shape=jax.ShapeDtypeStruct((NP, DW), jnp.float32),
    )(p1[0], p1[1], hs1, d0, d1, W2, b1r)

    p2 = _agg_kernel(hs2, src, dst)

    out = pl.pallas_call(
        _tc3_body,
        grid=(_GRID,),
        in_specs=[_row_spec(DW), _row_spec(DW), _row_spec(DW),
                  _row_spec(DW), _row_spec(DW),
                  _full_spec(W3.shape), _full_spec(b2r.shape),
                  _full_spec(b3r.shape)],
        out_specs=_row_spec(W3.shape[1]),
        out_shape=jax.ShapeDtypeStruct((NP, W3.shape[1]), jnp.float32),
    )(p2[0], p2[1], hs2, d0, d1, W3, b2r, b3r)

    return out[:N_NODES]


# speedup vs baseline: 2.9567x; 1.2157x over previous
import functools

import jax
import jax.numpy as jnp
from jax import lax
from jax.experimental import pallas as pl
from jax.experimental.pallas import tpu as pltpu
from jax.experimental.pallas import tpu_sc as plsc

N_NODES = 10000
NP = 10240
N_EDGES = 320000
CHUNK = 128
N_CHUNKS = 2560
EP = N_CHUNKS * CHUNK
CPT = 80
DH = 64
DW = 128
ROWS_PER_TILE = NP // 16

_mesh = plsc.VectorSubcoreMesh(core_axis_name="c", subcore_axis_name="s")


def _deg_body(dstc_hbm, out_hbm, dst_v, ones_v, deg_sh):
    c = lax.axis_index("c")
    s = lax.axis_index("s")
    wid = c * 16 + s

    def fill_zero(i, carry):
        for k in range(DW // 16):
            ones_v[i, pl.ds(k * 16, 16)] = jnp.zeros((16,), jnp.float32)
        return carry

    lax.fori_loop(0, CHUNK, fill_zero, 0)
    for q in range(ROWS_PER_TILE // CHUNK):
        pltpu.sync_copy(
            ones_v, deg_sh.at[pl.ds(s * ROWS_PER_TILE + q * CHUNK, CHUNK)])

    def fill_ones(i, carry):
        for k in range(DW // 16):
            ones_v[i, pl.ds(k * 16, 16)] = jnp.ones((16,), jnp.float32)
        return carry

    lax.fori_loop(0, CHUNK, fill_ones, 0)
    pltpu.sync_copy(dstc_hbm.at[pl.ds(wid * CPT, CPT)], dst_v)
    plsc.subcore_barrier()

    def body(j, carry):
        pltpu.sync_copy(ones_v, deg_sh.at[dst_v.at[j]], add=True)
        return carry

    lax.fori_loop(0, CPT, body, 0)
    plsc.subcore_barrier()
    pltpu.sync_copy(deg_sh.at[pl.ds(s * ROWS_PER_TILE, ROWS_PER_TILE)],
                    out_hbm.at[c, pl.ds(s * ROWS_PER_TILE, ROWS_PER_TILE)])


_deg_kernel = functools.partial(
    pl.kernel,
    out_type=jax.ShapeDtypeStruct((2, NP, DW), jnp.float32),
    mesh=_mesh,
    scratch_types=[
        pltpu.VMEM((CPT, CHUNK), jnp.int32),
        pltpu.VMEM((CHUNK, DW), jnp.float32),
        pltpu.VMEM_SHARED((NP, DW), jnp.float32),
    ],
)(_deg_body)


HCPT = CPT // 2
HPAIRS = HCPT // 2


def _agg_body(hs_hbm, srcc_hbm, dstc_hbm, out_hbm,
              src_v, dst_v, buf0, buf1, acc_sh, gsem, ssem):
    c = lax.axis_index("c")
    s = lax.axis_index("s")
    wid = c * 16 + s

    def fill_zero(i, carry):
        for k in range(DW // 16):
            buf0[i, pl.ds(k * 16, 16)] = jnp.zeros((16,), jnp.float32)
        return carry

    lax.fori_loop(0, CHUNK, fill_zero, 0)
    for q in range(ROWS_PER_TILE // CHUNK):
        pltpu.sync_copy(
            buf0, acc_sh.at[pl.ds(s * ROWS_PER_TILE + q * CHUNK, CHUNK)])
    plsc.subcore_barrier()

    def g_start(k, buf):
        pltpu.async_copy(hs_hbm.at[src_v.at[k]], buf, gsem)

    def g_wait(buf):
        pltpu.make_async_copy(hs_hbm.at[pl.ds(0, CHUNK)], buf, gsem).wait()

    def s_start(k, buf):
        pltpu.async_copy(buf, acc_sh.at[dst_v.at[k]], ssem, add=True)

    def s_wait(buf):
        pltpu.make_async_copy(hs_hbm.at[pl.ds(0, CHUNK)], buf, ssem).wait()

    for h in range(2):
        base = wid * CPT + h * HCPT
        pltpu.sync_copy(srcc_hbm.at[pl.ds(base, HCPT)], src_v)
        pltpu.sync_copy(dstc_hbm.at[pl.ds(base, HCPT)], dst_v)
        g_start(0, buf0)

        def body(i, carry):
            a = 2 * i
            g_wait(buf0)
            pl.when(i > 0)(lambda: s_wait(buf1))
            g_start(a + 1, buf1)
            s_start(a, buf0)
            g_wait(buf1)
            s_wait(buf0)
            pl.when(i < HPAIRS - 1)(lambda: g_start(a + 2, buf0))
            s_start(a + 1, buf1)
            return carry

        lax.fori_loop(0, HPAIRS, body, 0)
        s_wait(buf1)

    plsc.subcore_barrier()
    pltpu.sync_copy(acc_sh.at[pl.ds(s * ROWS_PER_TILE, ROWS_PER_TILE)],
                    out_hbm.at[c, pl.ds(s * ROWS_PER_TILE, ROWS_PER_TILE)])


_agg_kernel = functools.partial(
    pl.kernel,
    out_type=jax.ShapeDtypeStruct((2, NP, DW), jnp.float32),
    mesh=_mesh,
    scratch_types=[
        pltpu.VMEM((HCPT, CHUNK), jnp.int32),
        pltpu.VMEM((HCPT, CHUNK), jnp.int32),
        pltpu.VMEM((CHUNK, DW), jnp.float32),
        pltpu.VMEM((CHUNK, DW), jnp.float32),
        pltpu.VMEM_SHARED((NP, DW), jnp.float32),
        pltpu.SemaphoreType.DMA,
        pltpu.SemaphoreType.DMA,
    ],
)(_agg_body)


_GRID = 8
_BR = NP // _GRID


def _dinv(d0, d1):
    return lax.rsqrt(d0[:, :DH] + d1[:, :DH] + 1.0)


def _padw(h):
    return jnp.concatenate([h, jnp.zeros_like(h)], axis=1)


def _tc1_body(x_ref, w1_ref, d0_ref, d1_ref, hs_ref):
    dinv = _dinv(d0_ref[...], d1_ref[...])
    h = jnp.dot(x_ref[...], w1_ref[...], preferred_element_type=jnp.float32)
    hs_ref[...] = _padw(h * dinv)


def _tc2_body(p0_ref, p1_ref, hs1_ref, d0_ref, d1_ref, w2_ref, b1_ref, hs2_ref):
    dinv = _dinv(d0_ref[...], d1_ref[...])
    tot = (p0_ref[...] + p1_ref[...] + hs1_ref[...])[:, :DH]
    o1 = jnp.maximum(tot * dinv + b1_ref[...], 0.0)
    hs2_ref[...] = _padw(
        jnp.dot(o1, w2_ref[...], preferred_element_type=jnp.float32) * dinv)


def _tc3_body(p0_ref, p1_ref, hs2_ref, d0_ref, d1_ref, w3_ref, b2_ref, b3_ref,
              out_ref):
    dinv = _dinv(d0_ref[...], d1_ref[...])
    tot = (p0_ref[...] + p1_ref[...] + hs2_ref[...])[:, :DH]
    o2 = jnp.maximum(tot * dinv + b2_ref[...], 0.0)
    out_ref[...] = jnp.dot(
        o2, w3_ref[...], preferred_element_type=jnp.float32) + b3_ref[...]


def _row_spec(width):
    return pl.BlockSpec((_BR, width), lambda i: (i, 0))


def _full_spec(shape):
    return pl.BlockSpec(shape, lambda i: (0,) * len(shape))


def kernel(x, edge_index, W1, b1, W2, b2, W3, b3):
    pad_idx = N_NODES + (
        jnp.arange(EP - N_EDGES, dtype=jnp.int32) % (NP - N_NODES))
    src = jnp.concatenate(
        [edge_index[0], pad_idx]).reshape(N_CHUNKS, CHUNK)
    dst = jnp.concatenate(
        [edge_index[1], pad_idx]).reshape(N_CHUNKS, CHUNK)
    xp = jnp.pad(x, ((0, NP - N_NODES), (0, 0)))
    b1r = b1.reshape(1, DH)
    b2r = b2.reshape(1, DH)
    b3r = b3.reshape(1, W3.shape[1])

    degb = _deg_kernel(dst)
    d0, d1 = degb[0], degb[1]

    hs1 = pl.pallas_call(
        _tc1_body,
        grid=(_GRID,),
        in_specs=[_row_spec(x.shape[1]), _full_spec(W1.shape),
                  _row_spec(DW), _row_spec(DW)],
        out_specs=_row_spec(DW),
        out_shape=jax.ShapeDtypeStruct((NP, DW), jnp.float32),
    )(xp, W1, d0, d1)

    p1 = _agg_kernel(hs1, src, dst)

    hs2 = pl.pallas_call(
        _tc2_body,
        grid=(_GRID,),
        in_specs=[_row_spec(DW), _row_spec(DW), _row_spec(DW),
                  _row_spec(DW), _row_spec(DW),
                  _full_spec(W2.shape), _full_spec(b1r.shape)],
        out_specs=_row_spec(DW),
        out_shape=jax.ShapeDtypeStruct((NP, DW), jnp.float32),
    )(p1[0], p1[1], hs1, d0, d1, W2, b1r)

    p2 = _agg_kernel(hs2, src, dst)

    out = pl.pallas_call(
        _tc3_body,
        grid=(_GRID,),
        in_specs=[_row_spec(DW), _row_spec(DW), _row_spec(DW),
                  _row_spec(DW), _row_spec(DW),
                  _full_spec(W3.shape), _full_spec(b2r.shape),
                  _full_spec(b3r.shape)],
        out_specs=_row_spec(W3.shape[1]),
        out_shape=jax.ShapeDtypeStruct((NP, W3.shape[1]), jnp.float32),
    )(p2[0], p2[1], hs2, d0, d1, W3, b2r, b3r)

    return out[:N_NODES]


# speedup vs baseline: 3.3132x; 1.1206x over previous
import functools

import numpy as np

import jax
import jax.numpy as jnp
from jax import lax
from jax.experimental import pallas as pl
from jax.experimental.pallas import tpu as pltpu
from jax.experimental.pallas import tpu_sc as plsc

N_NODES = 10000
NP = 10240
N_EDGES = 320000
CHUNK = 128
N_CHUNKS = 2560
EP = N_CHUNKS * CHUNK
CPT = 80
DH = 64
DW = 128
ROWS_PER_TILE = NP // 16

_PAD_IDX = (np.arange(EP - N_EDGES) % (NP - N_NODES) + N_NODES).astype(np.int32)

_mesh = plsc.VectorSubcoreMesh(core_axis_name="c", subcore_axis_name="s")


def _deg_body(dstc_hbm, out_hbm, dst_v, hist_v, tmp_v, acc_v, red_v, hist_sh):
    c = lax.axis_index("c")
    s = lax.axis_index("s")
    wid = c * 16 + s
    ones16 = jnp.ones((16,), jnp.float32)

    pltpu.sync_copy(dstc_hbm.at[pl.ds(wid * CPT, CPT)], dst_v)

    def fill_zero(i, carry):
        hist_v[pl.ds(i * 16, 16)] = jnp.zeros((16,), jnp.float32)
        return carry

    lax.fori_loop(0, NP // 16, fill_zero, 0)

    def body(r, carry):
        for k in range(CHUNK // 16):
            idx16 = dst_v[r, pl.ds(k * 16, 16)]
            plsc.addupdate_scatter(hist_v, [idx16], ones16)
        return carry

    lax.fori_loop(0, CPT, body, 0)

    pltpu.sync_copy(hist_v, hist_sh.at[pl.ds(s * NP, NP)])
    plsc.subcore_barrier()

    def zero_acc(i, carry):
        acc_v[pl.ds(i * 16, 16)] = jnp.zeros((16,), jnp.float32)
        return carry

    lax.fori_loop(0, ROWS_PER_TILE // 16, zero_acc, 0)
    for t in range(16):
        pltpu.sync_copy(
            hist_sh.at[pl.ds(t * NP + s * ROWS_PER_TILE, ROWS_PER_TILE)],
            tmp_v)

        def add_part(i, carry):
            sl = pl.ds(i * 16, 16)
            acc_v[sl] = acc_v[sl] + tmp_v[sl]
            return carry

        lax.fori_loop(0, ROWS_PER_TILE // 16, add_part, 0)

    def widen(g, carry):
        vec = acc_v[pl.ds(g * 16, 16)]
        for l in range(16):
            row = jnp.full((16,), vec[l], jnp.float32)
            for k in range(DW // 16):
                red_v[g * 16 + l, pl.ds(k * 16, 16)] = row
        return carry

    lax.fori_loop(0, ROWS_PER_TILE // 16, widen, 0)
    pltpu.sync_copy(red_v,
                    out_hbm.at[c, pl.ds(s * ROWS_PER_TILE, ROWS_PER_TILE)])


_deg_kernel = functools.partial(
    pl.kernel,
    out_type=jax.ShapeDtypeStruct((2, NP, DW), jnp.float32),
    mesh=_mesh,
    scratch_types=[
        pltpu.VMEM((CPT, CHUNK), jnp.int32),
        pltpu.VMEM((NP,), jnp.float32),
        pltpu.VMEM((ROWS_PER_TILE,), jnp.float32),
        pltpu.VMEM((ROWS_PER_TILE,), jnp.float32),
        pltpu.VMEM((ROWS_PER_TILE, DW), jnp.float32),
        pltpu.VMEM_SHARED((16 * NP,), jnp.float32),
    ],
    compiler_params=pltpu.CompilerParams(needs_layout_passes=False),
)(_deg_body)


HCPT = CPT // 2
HPAIRS = HCPT // 2


def _agg_body(hs_hbm, srcc_hbm, dstc_hbm, out_hbm,
              src_v, dst_v, buf0, buf1, acc_sh, gsem, ssem):
    c = lax.axis_index("c")
    s = lax.axis_index("s")
    wid = c * 16 + s

    def fill_zero(i, carry):
        for k in range(DW // 16):
            buf0[i, pl.ds(k * 16, 16)] = jnp.zeros((16,), jnp.float32)
        return carry

    lax.fori_loop(0, CHUNK, fill_zero, 0)
    for q in range(ROWS_PER_TILE // CHUNK):
        pltpu.sync_copy(
            buf0, acc_sh.at[pl.ds(s * ROWS_PER_TILE + q * CHUNK, CHUNK)])
    plsc.subcore_barrier()

    def g_start(k, buf):
        pltpu.async_copy(hs_hbm.at[src_v.at[k]], buf, gsem)

    def g_wait(buf):
        pltpu.make_async_copy(hs_hbm.at[pl.ds(0, CHUNK)], buf, gsem).wait()

    def s_start(k, buf):
        pltpu.async_copy(buf, acc_sh.at[dst_v.at[k]], ssem, add=True)

    def s_wait(buf):
        pltpu.make_async_copy(hs_hbm.at[pl.ds(0, CHUNK)], buf, ssem).wait()

    for h in range(2):
        base = wid * CPT + h * HCPT
        pltpu.sync_copy(srcc_hbm.at[pl.ds(base, HCPT)], src_v)
        pltpu.sync_copy(dstc_hbm.at[pl.ds(base, HCPT)], dst_v)
        g_start(0, buf0)

        def body(i, carry):
            a = 2 * i
            g_wait(buf0)
            pl.when(i > 0)(lambda: s_wait(buf1))
            g_start(a + 1, buf1)
            s_start(a, buf0)
            g_wait(buf1)
            s_wait(buf0)
            pl.when(i < HPAIRS - 1)(lambda: g_start(a + 2, buf0))
            s_start(a + 1, buf1)
            return carry

        lax.fori_loop(0, HPAIRS, body, 0)
        s_wait(buf1)

    plsc.subcore_barrier()
    pltpu.sync_copy(acc_sh.at[pl.ds(s * ROWS_PER_TILE, ROWS_PER_TILE)],
                    out_hbm.at[c, pl.ds(s * ROWS_PER_TILE, ROWS_PER_TILE)])


_agg_kernel = functools.partial(
    pl.kernel,
    out_type=jax.ShapeDtypeStruct((2, NP, DW), jnp.float32),
    mesh=_mesh,
    scratch_types=[
        pltpu.VMEM((HCPT, CHUNK), jnp.int32),
        pltpu.VMEM((HCPT, CHUNK), jnp.int32),
        pltpu.VMEM((CHUNK, DW), jnp.float32),
        pltpu.VMEM((CHUNK, DW), jnp.float32),
        pltpu.VMEM_SHARED((NP, DW), jnp.float32),
        pltpu.SemaphoreType.DMA,
        pltpu.SemaphoreType.DMA,
    ],
)(_agg_body)


_GRID = 8
_BR = NP // _GRID


def _dinv(d0, d1):
    return lax.rsqrt(d0[:, :DH] + d1[:, :DH] + 1.0)


def _padw(h):
    return jnp.concatenate([h, jnp.zeros_like(h)], axis=1)


def _tc1_body(x_ref, w1_ref, d0_ref, d1_ref, hs_ref):
    dinv = _dinv(d0_ref[...], d1_ref[...])
    h = jnp.dot(x_ref[...], w1_ref[...], preferred_element_type=jnp.float32)
    hs_ref[...] = _padw(h * dinv)


def _tc2_body(p0_ref, p1_ref, hs1_ref, d0_ref, d1_ref, w2_ref, b1_ref, hs2_ref):
    dinv = _dinv(d0_ref[...], d1_ref[...])
    tot = (p0_ref[...] + p1_ref[...] + hs1_ref[...])[:, :DH]
    o1 = jnp.maximum(tot * dinv + b1_ref[...], 0.0)
    hs2_ref[...] = _padw(
        jnp.dot(o1, w2_ref[...], preferred_element_type=jnp.float32) * dinv)


def _tc3_body(p0_ref, p1_ref, hs2_ref, d0_ref, d1_ref, w3_ref, b2_ref, b3_ref,
              out_ref):
    dinv = _dinv(d0_ref[...], d1_ref[...])
    tot = (p0_ref[...] + p1_ref[...] + hs2_ref[...])[:, :DH]
    o2 = jnp.maximum(tot * dinv + b2_ref[...], 0.0)
    out_ref[...] = jnp.dot(
        o2, w3_ref[...], preferred_element_type=jnp.float32) + b3_ref[...]


def _row_spec(width):
    return pl.BlockSpec((_BR, width), lambda i: (i, 0))


def _full_spec(shape):
    return pl.BlockSpec(shape, lambda i: (0,) * len(shape))


def kernel(x, edge_index, W1, b1, W2, b2, W3, b3):
    pad_idx = jnp.asarray(_PAD_IDX)
    src = jnp.concatenate(
        [edge_index[0], pad_idx]).reshape(N_CHUNKS, CHUNK)
    dst = jnp.concatenate(
        [edge_index[1], pad_idx]).reshape(N_CHUNKS, CHUNK)
    xp = jnp.pad(x, ((0, NP - N_NODES), (0, 0)))
    b1r = b1.reshape(1, DH)
    b2r = b2.reshape(1, DH)
    b3r = b3.reshape(1, W3.shape[1])

    degb = _deg_kernel(dst)
    d0, d1 = degb[0], degb[1]

    hs1 = pl.pallas_call(
        _tc1_body,
        grid=(_GRID,),
        in_specs=[_row_spec(x.shape[1]), _full_spec(W1.shape),
                  _row_spec(DW), _row_spec(DW)],
        out_specs=_row_spec(DW),
        out_shape=jax.ShapeDtypeStruct((NP, DW), jnp.float32),
    )(xp, W1, d0, d1)

    p1 = _agg_kernel(hs1, src, dst)

    hs2 = pl.pallas_call(
        _tc2_body,
        grid=(_GRID,),
        in_specs=[_row_spec(DW), _row_spec(DW), _row_spec(DW),
                  _row_spec(DW), _row_spec(DW),
                  _full_spec(W2.shape), _full_spec(b1r.shape)],
        out_specs=_row_spec(DW),
        out_shape=jax.ShapeDtypeStruct((NP, DW), jnp.float32),
    )(p1[0], p1[1], hs1, d0, d1, W2, b1r)

    p2 = _agg_kernel(hs2, src, dst)

    out = pl.pallas_call(
        _tc3_body,
        grid=(_GRID,),
        in_specs=[_row_spec(DW), _row_spec(DW), _row_spec(DW),
                  _row_spec(DW), _row_spec(DW),
                  _full_spec(W3.shape), _full_spec(b2r.shape),
                  _full_spec(b3r.shape)],
        out_specs=_row_spec(W3.shape[1]),
        out_shape=jax.ShapeDtypeStruct((NP, W3.shape[1]), jnp.float32),
    )(p2[0], p2[1], hs2, d0, d1, W3, b2r, b3r)

    return out[:N_NODES]


# speedup vs baseline: 3.7752x; 1.1394x over previous
import functools

import numpy as np

import jax
import jax.numpy as jnp
from jax import lax
from jax.experimental import pallas as pl
from jax.experimental.pallas import tpu as pltpu
from jax.experimental.pallas import tpu_sc as plsc

N_NODES = 10000
NP = 10240
N_EDGES = 320000
CHUNK = 128
N_CHUNKS = 2560
EP = N_CHUNKS * CHUNK
CPT = 80
DH = 64
DW = 64
ROWS_PER_TILE = NP // 16

_PAD_IDX = (np.arange(EP - N_EDGES) % (NP - N_NODES) + N_NODES).astype(np.int32)

_mesh = plsc.VectorSubcoreMesh(core_axis_name="c", subcore_axis_name="s")


def _deg_body(dstc_hbm, out_hbm, dst_v, hist_v, tmp_v, acc_v, red_v, hist_sh):
    c = lax.axis_index("c")
    s = lax.axis_index("s")
    wid = c * 16 + s
    ones16 = jnp.ones((16,), jnp.float32)

    pltpu.sync_copy(dstc_hbm.at[pl.ds(wid * CPT, CPT)], dst_v)

    def fill_zero(i, carry):
        hist_v[pl.ds(i * 16, 16)] = jnp.zeros((16,), jnp.float32)
        return carry

    lax.fori_loop(0, NP // 16, fill_zero, 0)

    def body(r, carry):
        for k in range(CHUNK // 16):
            idx16 = dst_v[r, pl.ds(k * 16, 16)]
            plsc.addupdate_scatter(hist_v, [idx16], ones16)
        return carry

    lax.fori_loop(0, CPT, body, 0)

    pltpu.sync_copy(hist_v, hist_sh.at[pl.ds(s * NP, NP)])
    plsc.subcore_barrier()

    def zero_acc(i, carry):
        acc_v[pl.ds(i * 16, 16)] = jnp.zeros((16,), jnp.float32)
        return carry

    lax.fori_loop(0, ROWS_PER_TILE // 16, zero_acc, 0)
    for t in range(16):
        pltpu.sync_copy(
            hist_sh.at[pl.ds(t * NP + s * ROWS_PER_TILE, ROWS_PER_TILE)],
            tmp_v)

        def add_part(i, carry):
            sl = pl.ds(i * 16, 16)
            acc_v[sl] = acc_v[sl] + tmp_v[sl]
            return carry

        lax.fori_loop(0, ROWS_PER_TILE // 16, add_part, 0)

    def widen(g, carry):
        vec = acc_v[pl.ds(g * 16, 16)]
        for l in range(16):
            row = jnp.full((16,), vec[l], jnp.float32)
            for k in range(DW // 16):
                red_v[g * 16 + l, pl.ds(k * 16, 16)] = row
        return carry

    lax.fori_loop(0, ROWS_PER_TILE // 16, widen, 0)
    pltpu.sync_copy(red_v,
                    out_hbm.at[c, pl.ds(s * ROWS_PER_TILE, ROWS_PER_TILE)])


_deg_kernel = functools.partial(
    pl.kernel,
    out_type=jax.ShapeDtypeStruct((2, NP, DW), jnp.float32),
    mesh=_mesh,
    scratch_types=[
        pltpu.VMEM((CPT, CHUNK), jnp.int32),
        pltpu.VMEM((NP,), jnp.float32),
        pltpu.VMEM((ROWS_PER_TILE,), jnp.float32),
        pltpu.VMEM((ROWS_PER_TILE,), jnp.float32),
        pltpu.VMEM((ROWS_PER_TILE, DW), jnp.float32),
        pltpu.VMEM_SHARED((16 * NP,), jnp.float32),
    ],
    compiler_params=pltpu.CompilerParams(
        needs_layout_passes=False, use_tc_tiling_on_sc=False),
)(_deg_body)


HCPT = CPT // 2
HPAIRS = HCPT // 2


def _agg_body(hs_hbm, srcc_hbm, dstc_hbm, out_hbm,
              src_v, dst_v, buf0, buf1, acc_sh, gsem, ssem):
    c = lax.axis_index("c")
    s = lax.axis_index("s")
    wid = c * 16 + s

    def fill_zero(i, carry):
        for k in range(DW // 16):
            buf0[i, pl.ds(k * 16, 16)] = jnp.zeros((16,), jnp.float32)
        return carry

    lax.fori_loop(0, CHUNK, fill_zero, 0)
    for q in range(ROWS_PER_TILE // CHUNK):
        pltpu.sync_copy(
            buf0, acc_sh.at[pl.ds(s * ROWS_PER_TILE + q * CHUNK, CHUNK)])
    plsc.subcore_barrier()

    def g_start(k, buf):
        pltpu.async_copy(hs_hbm.at[src_v.at[k]], buf, gsem)

    def g_wait(buf):
        pltpu.make_async_copy(hs_hbm.at[pl.ds(0, CHUNK)], buf, gsem).wait()

    def s_start(k, buf):
        pltpu.async_copy(buf, acc_sh.at[dst_v.at[k]], ssem, add=True)

    def s_wait(buf):
        pltpu.make_async_copy(hs_hbm.at[pl.ds(0, CHUNK)], buf, ssem).wait()

    for h in range(2):
        base = wid * CPT + h * HCPT
        pltpu.sync_copy(srcc_hbm.at[pl.ds(base, HCPT)], src_v)
        pltpu.sync_copy(dstc_hbm.at[pl.ds(base, HCPT)], dst_v)
        g_start(0, buf0)

        def body(i, carry):
            a = 2 * i
            g_wait(buf0)
            pl.when(i > 0)(lambda: s_wait(buf1))
            g_start(a + 1, buf1)
            s_start(a, buf0)
            g_wait(buf1)
            s_wait(buf0)
            pl.when(i < HPAIRS - 1)(lambda: g_start(a + 2, buf0))
            s_start(a + 1, buf1)
            return carry

        lax.fori_loop(0, HPAIRS, body, 0)
        s_wait(buf1)

    plsc.subcore_barrier()
    pltpu.sync_copy(acc_sh.at[pl.ds(s * ROWS_PER_TILE, ROWS_PER_TILE)],
                    out_hbm.at[c, pl.ds(s * ROWS_PER_TILE, ROWS_PER_TILE)])


_agg_kernel = functools.partial(
    pl.kernel,
    out_type=jax.ShapeDtypeStruct((2, NP, DW), jnp.float32),
    mesh=_mesh,
    scratch_types=[
        pltpu.VMEM((HCPT, CHUNK), jnp.int32),
        pltpu.VMEM((HCPT, CHUNK), jnp.int32),
        pltpu.VMEM((CHUNK, DW), jnp.float32),
        pltpu.VMEM((CHUNK, DW), jnp.float32),
        pltpu.VMEM_SHARED((NP, DW), jnp.float32),
        pltpu.SemaphoreType.DMA,
        pltpu.SemaphoreType.DMA,
    ],
    compiler_params=pltpu.CompilerParams(
        needs_layout_passes=False, use_tc_tiling_on_sc=False),
)(_agg_body)


_GRID = 8
_BR = NP // _GRID


def _dinv(d0, d1):
    return lax.rsqrt(d0 + d1 + 1.0)


def _tc1_body(x_ref, w1_ref, d0_ref, d1_ref, hs_ref):
    dinv = _dinv(d0_ref[...], d1_ref[...])
    h = jnp.dot(x_ref[...], w1_ref[...], preferred_element_type=jnp.float32)
    hs_ref[...] = h * dinv


def _tc2_body(p0_ref, p1_ref, hs1_ref, d0_ref, d1_ref, w2_ref, b1_ref, hs2_ref):
    dinv = _dinv(d0_ref[...], d1_ref[...])
    tot = p0_ref[...] + p1_ref[...] + hs1_ref[...]
    o1 = jnp.maximum(tot * dinv + b1_ref[...], 0.0)
    hs2_ref[...] = jnp.dot(
        o1, w2_ref[...], preferred_element_type=jnp.float32) * dinv


def _tc3_body(p0_ref, p1_ref, hs2_ref, d0_ref, d1_ref, w3_ref, b2_ref, b3_ref,
              out_ref):
    dinv = _dinv(d0_ref[...], d1_ref[...])
    tot = p0_ref[...] + p1_ref[...] + hs2_ref[...]
    o2 = jnp.maximum(tot * dinv + b2_ref[...], 0.0)
    out_ref[...] = jnp.dot(
        o2, w3_ref[...], preferred_element_type=jnp.float32) + b3_ref[...]


def _row_spec(width):
    return pl.BlockSpec((_BR, width), lambda i: (i, 0))


def _full_spec(shape):
    return pl.BlockSpec(shape, lambda i: (0,) * len(shape))


def kernel(x, edge_index, W1, b1, W2, b2, W3, b3):
    pad_idx = jnp.asarray(_PAD_IDX)
    src = jnp.concatenate(
        [edge_index[0], pad_idx]).reshape(N_CHUNKS, CHUNK)
    dst = jnp.concatenate(
        [edge_index[1], pad_idx]).reshape(N_CHUNKS, CHUNK)
    xp = jnp.pad(x, ((0, NP - N_NODES), (0, 0)))
    b1r = b1.reshape(1, DH)
    b2r = b2.reshape(1, DH)
    b3r = b3.reshape(1, W3.shape[1])

    degb = _deg_kernel(dst)
    d0, d1 = degb[0], degb[1]

    hs1 = pl.pallas_call(
        _tc1_body,
        grid=(_GRID,),
        in_specs=[_row_spec(x.shape[1]), _full_spec(W1.shape),
                  _row_spec(DW), _row_spec(DW)],
        out_specs=_row_spec(DW),
        out_shape=jax.ShapeDtypeStruct((NP, DW), jnp.float32),
    )(xp, W1, d0, d1)

    p1 = _agg_kernel(hs1, src, dst)

    hs2 = pl.pallas_call(
        _tc2_body,
        grid=(_GRID,),
        in_specs=[_row_spec(DW), _row_spec(DW), _row_spec(DW),
                  _row_spec(DW), _row_spec(DW),
                  _full_spec(W2.shape), _full_spec(b1r.shape)],
        out_specs=_row_spec(DW),
        out_shape=jax.ShapeDtypeStruct((NP, DW), jnp.float32),
    )(p1[0], p1[1], hs1, d0, d1, W2, b1r)

    p2 = _agg_kernel(hs2, src, dst)

    out = pl.pallas_call(
        _tc3_body,
        grid=(_GRID,),
        in_specs=[_row_spec(DW), _row_spec(DW), _row_spec(DW),
                  _row_spec(DW), _row_spec(DW),
                  _full_spec(W3.shape), _full_spec(b2r.shape),
                  _full_spec(b3r.shape)],
        out_specs=_row_spec(W3.shape[1]),
        out_shape=jax.ShapeDtypeStruct((NP, W3.shape[1]), jnp.float32),
    )(p2[0], p2[1], hs2, d0, d1, W3, b2r, b3r)

    return out[:N_NODES]
